# Initial kernel scaffold; baseline (speedup 1.0000x reference)
import jax
import jax.numpy as jnp
from jax.experimental import pallas as pl


def kernel(paths_mm, paths_dd, paths_md, samples, miRNA, disease, Wm, Wd, pw1, pw2, fcW, mW0, mb0, mW1, mb1, mW2, mb2):
    raise NotImplementedError("write your pallas kernel here")


# speedup vs baseline: 3.1394x; 3.1394x over previous
import functools

import jax
import jax.numpy as jnp
from jax import lax
from jax.experimental import pallas as pl
from jax.experimental.pallas import tpu as pltpu
from jax.experimental.pallas import tpu_sc as plsc

Nm, Nd, D = 4096, 4096, 128
P, L1, L2, NL = 8, 4, 8, 2
NS = 16384
ALPHA = 0.1

N2 = Nm + Nd
NW = 32
NPW = N2 // NW
GR = 128
NCHUNK = NPW // GR
VL = 16
NSPW = NS // NW


def _matmul_kernel(a_ref, b_ref, o_ref):
    o_ref[...] = jnp.dot(a_ref[...], b_ref[...],
                         preferred_element_type=jnp.float32)


def _matmul(a, b):
    m, k = a.shape
    _, n = b.shape
    bm = 256
    return pl.pallas_call(
        _matmul_kernel,
        grid=(m // bm,),
        in_specs=[
            pl.BlockSpec((bm, k), lambda i: (i, 0)),
            pl.BlockSpec((k, n), lambda i: (0, 0)),
        ],
        out_specs=pl.BlockSpec((bm, n), lambda i: (i, 0)),
        out_shape=jax.ShapeDtypeStruct((m, n), jnp.float32),
    )(a, b)


def _fc_kernel(r_ref, w_ref, h_ref, o_ref):
    f = jnp.dot(r_ref[...], w_ref[...], preferred_element_type=jnp.float32)
    o_ref[...] = ALPHA * h_ref[...] + jnp.maximum(f, 0.0)


def _fc(r, wt, h):
    bm = 1024
    return pl.pallas_call(
        _fc_kernel,
        grid=(N2 // bm,),
        in_specs=[
            pl.BlockSpec((bm, D), lambda i: (i, 0)),
            pl.BlockSpec((D, D), lambda i: (0, 0)),
            pl.BlockSpec((bm, D), lambda i: (i, 0)),
        ],
        out_specs=pl.BlockSpec((bm, D), lambda i: (i, 0)),
        out_shape=jax.ShapeDtypeStruct((N2, D), jnp.float32),
    )(r, wt, h)


def _score_kernel(fa_ref, fb_ref, w_ref, b_ref, o_ref):
    dn = (((1,), (1,)), ((), ()))
    pa = lax.dot_general(w_ref[...], fa_ref[...], dn,
                         preferred_element_type=jnp.float32)
    pb = lax.dot_general(w_ref[...], fb_ref[...], dn,
                         preferred_element_type=jnp.float32)
    b = b_ref[0]
    o_ref[0:1, :] = pa[0:1, 0:Nm] + pb[1:2, 0:Nm] + b
    o_ref[1:2, :] = pa[2:3, Nm:N2] + pb[3:4, Nm:N2] + b


def _score(fa, fb, w4, bvec):
    return pl.pallas_call(
        _score_kernel,
        in_specs=[
            pl.BlockSpec((N2, D), lambda: (0, 0)),
            pl.BlockSpec((N2, D), lambda: (0, 0)),
            pl.BlockSpec((4, D), lambda: (0, 0)),
            pl.BlockSpec(memory_space=pltpu.SMEM),
        ],
        out_specs=pl.BlockSpec((2, Nm), lambda: (0, 0)),
        out_shape=jax.ShapeDtypeStruct((2, Nm), jnp.float32),
    )(fa, fb, w4, bvec)


@functools.lru_cache(maxsize=None)
def _make_gather_combine(nstream):
    pl_per_chunk = nstream // NCHUNK
    shift = pl_per_chunk.bit_length() - 1
    mesh = plsc.VectorSubcoreMesh(core_axis_name="c", subcore_axis_name="s")

    @functools.partial(
        pl.kernel,
        out_type=jax.ShapeDtypeStruct((N2, D), jnp.float32),
        mesh=mesh,
        scratch_types=[
            pltpu.VMEM((nstream, GR), jnp.int32),
            pltpu.VMEM((nstream, D), jnp.float32),
            pltpu.VMEM((GR, D), jnp.float32),
            pltpu.VMEM((GR, D), jnp.float32),
            pltpu.VMEM((NPW, D), jnp.float32),
            pltpu.SemaphoreType.DMA,
            pltpu.SemaphoreType.DMA,
        ],
    )
    def k(table, idx_hbm, w_hbm, out_hbm, idx_v, w_v, buf0, buf1, acc,
          sem0, sem1):
        wid = lax.axis_index("s") * 2 + lax.axis_index("c")
        base = wid * NPW
        pltpu.sync_copy(idx_hbm.at[wid], idx_v)
        pltpu.sync_copy(w_hbm, w_v)

        zero = jnp.zeros((VL,), jnp.float32)

        def zrow(r, c):
            for kk in range(D // VL):
                acc[r, pl.ds(kk * VL, VL)] = zero
            return c

        lax.fori_loop(0, NPW, zrow, 0)

        def fire(s, buf, sem):
            pltpu.make_async_copy(table.at[idx_v.at[s]], buf, sem).start()

        def drain(buf, sem):
            pltpu.make_async_copy(table.at[idx_v.at[0]], buf, sem).wait()

        def accum(s, buf):
            nb = lax.shift_right_logical(s, shift) * GR
            wv = [w_v[s, pl.ds(kk * VL, VL)] for kk in range(D // VL)]

            def row(r, c):
                for kk in range(D // VL):
                    plsc.addupdate(
                        acc.at[nb + r, pl.ds(kk * VL, VL)],
                        buf[r, pl.ds(kk * VL, VL)] * wv[kk])
                return c

            lax.fori_loop(0, GR, row, 0)

        fire(0, buf0, sem0)

        def pair(i, c):
            s = 2 * i
            fire(s + 1, buf1, sem1)
            drain(buf0, sem0)
            accum(s, buf0)

            @pl.when(s + 2 < nstream)
            def _():
                fire(s + 2, buf0, sem0)

            drain(buf1, sem1)
            accum(s + 1, buf1)
            return c

        lax.fori_loop(0, nstream // 2, pair, 0)
        pltpu.sync_copy(acc, out_hbm.at[pl.ds(base, NPW)])

    return k


@functools.lru_cache(maxsize=None)
def _make_sample_kernel():
    mesh = plsc.VectorSubcoreMesh(core_axis_name="c", subcore_axis_name="s")

    @functools.partial(
        pl.kernel,
        out_type=jax.ShapeDtypeStruct((NS,), jnp.float32),
        mesh=mesh,
        scratch_types=[
            pltpu.VMEM((NSPW,), jnp.int32),
            pltpu.VMEM((NSPW,), jnp.int32),
            pltpu.VMEM((NSPW,), jnp.float32),
            pltpu.VMEM((NSPW,), jnp.float32),
            pltpu.VMEM((NSPW,), jnp.float32),
            pltpu.SemaphoreType.DMA,
            pltpu.SemaphoreType.DMA,
        ],
    )
    def k(sm_hbm, sd_hbm, s_hbm, out_hbm, s0, s1, v0, v1, ov, sem0, sem1):
        wid = lax.axis_index("s") * 2 + lax.axis_index("c")
        base = wid * NSPW
        pltpu.sync_copy(s_hbm.at[0, pl.ds(base, NSPW)], s0)
        pltpu.sync_copy(s_hbm.at[1, pl.ds(base, NSPW)], s1)
        c0 = pltpu.async_copy(sm_hbm.at[s0], v0, sem0)
        c1 = pltpu.async_copy(sd_hbm.at[s1], v1, sem1)
        c0.wait()
        c1.wait()

        def body(i, c):
            x = v0[pl.ds(i * VL, VL)] + v1[pl.ds(i * VL, VL)]
            ov[pl.ds(i * VL, VL)] = 1.0 / (1.0 + jnp.exp(-x))
            return c

        lax.fori_loop(0, NSPW // VL, body, 0)
        pltpu.sync_copy(ov, out_hbm.at[pl.ds(base, NSPW)])

    return k


def _relayout_idx(idx):
    p, _, l = idx.shape
    x = idx.astype(jnp.int32).transpose(1, 0, 2)
    x = x.reshape(NW, NCHUNK, GR, p * l)
    return x.transpose(0, 1, 3, 2).reshape(NW, NCHUNK * p * l, GR)


def _stream_weights(pw_l):
    w_full = jnp.tile(pw_l / float(P), (P, 1))
    return jnp.tile(w_full, (NCHUNK, 1))


def kernel(paths_mm, paths_dd, paths_md, samples, miRNA, disease, Wm, Wd,
           pw1, pw2, fcW, mW0, mb0, mW1, mb1, mW2, mb2):
    idx_a = _relayout_idx(
        jnp.concatenate([paths_mm, paths_dd + Nm], axis=1))
    idx_b = _relayout_idx(paths_md)
    ws_a = [_stream_weights(pw1[l]) for l in range(NL)]
    ws_b = [_stream_weights(pw2[l]) for l in range(NL)]
    wt = [(1.0 - ALPHA) * fcW[l].T for l in range(NL)]

    w512 = (mW2 @ mW1 @ mW0).reshape(4, D)
    bias = (mW2 @ (mW1 @ mb0 + mb1) + mb2).reshape(1)
    s_t = samples.astype(jnp.int32).T

    hm = _matmul(miRNA, Wm)
    hd = _matmul(disease, Wd)
    hcat = jnp.concatenate([hm, hd], axis=0)

    gather_a = _make_gather_combine(NCHUNK * P * L1)
    gather_b = _make_gather_combine(NCHUNK * P * L2)

    feats_a = hcat
    for l in range(NL):
        r = gather_a(feats_a, idx_a, ws_a[l])
        feats_a = _fc(r, wt[l], hcat)

    feats_b = hcat
    for l in range(NL):
        r = gather_b(feats_b, idx_b, ws_b[l])
        feats_b = _fc(r, wt[l], hcat)

    score2 = _score(feats_a, feats_b, w512, bias)
    out = _make_sample_kernel()(score2[0], score2[1], s_t)
    return out.reshape(NS, 1)


# speedup vs baseline: 10.9587x; 3.4907x over previous
import functools

import jax
import jax.numpy as jnp
from jax import lax
from jax.experimental import pallas as pl
from jax.experimental.pallas import tpu as pltpu
from jax.experimental.pallas import tpu_sc as plsc

Nm, Nd, D = 4096, 4096, 128
P, L1, L2, NL = 8, 4, 8, 2
NS = 16384
ALPHA = 0.1

N2 = Nm + Nd
NW = 32
NPW = N2 // NW
GR = 128
NCHUNK = NPW // GR
VL = 16
NSPW = NS // NW


def _matmul_kernel(a_ref, b_ref, o_ref):
    o_ref[...] = jnp.dot(a_ref[...], b_ref[...],
                         preferred_element_type=jnp.float32)


def _matmul(a, b):
    m, k = a.shape
    _, n = b.shape
    bm = 256
    return pl.pallas_call(
        _matmul_kernel,
        grid=(m // bm,),
        in_specs=[
            pl.BlockSpec((bm, k), lambda i: (i, 0)),
            pl.BlockSpec((k, n), lambda i: (0, 0)),
        ],
        out_specs=pl.BlockSpec((bm, n), lambda i: (i, 0)),
        out_shape=jax.ShapeDtypeStruct((m, n), jnp.float32),
    )(a, b)


def _fc_kernel(r_ref, w_ref, h_ref, o_ref):
    f = jnp.dot(r_ref[...], w_ref[...], preferred_element_type=jnp.float32)
    o_ref[...] = ALPHA * h_ref[...] + jnp.maximum(f, 0.0)


def _fc(r, wt, h):
    bm = 1024
    return pl.pallas_call(
        _fc_kernel,
        grid=(N2 // bm,),
        in_specs=[
            pl.BlockSpec((bm, D), lambda i: (i, 0)),
            pl.BlockSpec((D, D), lambda i: (0, 0)),
            pl.BlockSpec((bm, D), lambda i: (i, 0)),
        ],
        out_specs=pl.BlockSpec((bm, D), lambda i: (i, 0)),
        out_shape=jax.ShapeDtypeStruct((N2, D), jnp.float32),
    )(r, wt, h)


def _prescale_kernel(f_ref, w_ref, o_ref):
    lsel = pl.program_id(0)
    o_ref[...] = f_ref[...] * w_ref[pl.ds(lsel, 1), :]


def _prescale(feats, pw_s):
    nl = pw_s.shape[0]
    bm = 1024
    nb = N2 // bm
    return pl.pallas_call(
        _prescale_kernel,
        grid=(nl, nb),
        in_specs=[
            pl.BlockSpec((bm, D), lambda l, i: (i, 0)),
            pl.BlockSpec((nl, D), lambda l, i: (0, 0)),
        ],
        out_specs=pl.BlockSpec((bm, D), lambda l, i: (l * nb + i, 0)),
        out_shape=jax.ShapeDtypeStruct((nl * N2, D), jnp.float32),
    )(feats, pw_s)


def _score_kernel(fa_ref, fb_ref, w_ref, b_ref, o_ref):
    dn = (((1,), (1,)), ((), ()))
    pa = lax.dot_general(w_ref[...], fa_ref[...], dn,
                         preferred_element_type=jnp.float32)
    pb = lax.dot_general(w_ref[...], fb_ref[...], dn,
                         preferred_element_type=jnp.float32)
    b = b_ref[0]
    o_ref[0:1, :] = pa[0:1, 0:Nm] + pb[1:2, 0:Nm] + b
    o_ref[1:2, :] = pa[2:3, Nm:N2] + pb[3:4, Nm:N2] + b


def _score(fa, fb, w4, bvec):
    return pl.pallas_call(
        _score_kernel,
        in_specs=[
            pl.BlockSpec((N2, D), lambda: (0, 0)),
            pl.BlockSpec((N2, D), lambda: (0, 0)),
            pl.BlockSpec((4, D), lambda: (0, 0)),
            pl.BlockSpec(memory_space=pltpu.SMEM),
        ],
        out_specs=pl.BlockSpec((2, Nm), lambda: (0, 0)),
        out_shape=jax.ShapeDtypeStruct((2, Nm), jnp.float32),
    )(fa, fb, w4, bvec)


@functools.lru_cache(maxsize=None)
def _make_gather_combine(nstream, nl):
    pl_per_chunk = nstream // NCHUNK
    shift = pl_per_chunk.bit_length() - 1
    mesh = plsc.VectorSubcoreMesh(core_axis_name="c", subcore_axis_name="s")

    @functools.partial(
        pl.kernel,
        out_type=jax.ShapeDtypeStruct((N2, D), jnp.float32),
        mesh=mesh,
        scratch_types=[
            pltpu.VMEM((nstream, GR), jnp.int32),
            pltpu.VMEM((GR, D), jnp.float32),
            pltpu.VMEM((GR, D), jnp.float32),
            pltpu.VMEM((NPW, D), jnp.float32),
            pltpu.SemaphoreType.DMA,
            pltpu.SemaphoreType.DMA,
        ],
    )
    def k(table, idx_hbm, out_hbm, idx_v, buf0, buf1, acc, sem0, sem1):
        wid = lax.axis_index("s") * 2 + lax.axis_index("c")
        base = wid * NPW
        pltpu.sync_copy(idx_hbm.at[wid], idx_v)

        zero = jnp.zeros((VL,), jnp.float32)

        @functools.partial(plsc.parallel_loop, 0, NPW, unroll=4)
        def _(r):
            row = acc.at[r]
            for kk in range(D // VL):
                row[pl.ds(kk * VL, VL)] = zero

        def fire(s, buf, sem):
            pltpu.make_async_copy(table.at[idx_v.at[s]], buf, sem).start()

        def drain(buf, sem):
            pltpu.make_async_copy(table.at[idx_v.at[0]], buf, sem).wait()

        def accum(s, buf):
            nb = lax.shift_right_logical(s, shift) * GR

            @functools.partial(plsc.parallel_loop, 0, GR, unroll=4)
            def _(r):
                dst = acc.at[nb + r]
                src = buf.at[r]
                for kk in range(D // VL):
                    plsc.addupdate(dst.at[pl.ds(kk * VL, VL)],
                                   src[pl.ds(kk * VL, VL)])

        fire(0, buf0, sem0)

        def pair(i, c):
            s = 2 * i
            fire(s + 1, buf1, sem1)
            drain(buf0, sem0)
            accum(s, buf0)

            @pl.when(s + 2 < nstream)
            def _():
                fire(s + 2, buf0, sem0)

            drain(buf1, sem1)
            accum(s + 1, buf1)
            return c

        lax.fori_loop(0, nstream // 2, pair, 0)
        pltpu.sync_copy(acc, out_hbm.at[pl.ds(base, NPW)])

    return k


@functools.lru_cache(maxsize=None)
def _make_sample_kernel():
    mesh = plsc.VectorSubcoreMesh(core_axis_name="c", subcore_axis_name="s")

    @functools.partial(
        pl.kernel,
        out_type=jax.ShapeDtypeStruct((NS,), jnp.float32),
        mesh=mesh,
        scratch_types=[
            pltpu.VMEM((NSPW,), jnp.int32),
            pltpu.VMEM((NSPW,), jnp.int32),
            pltpu.VMEM((NSPW,), jnp.float32),
            pltpu.VMEM((NSPW,), jnp.float32),
            pltpu.VMEM((NSPW,), jnp.float32),
            pltpu.SemaphoreType.DMA,
            pltpu.SemaphoreType.DMA,
        ],
    )
    def k(sm_hbm, sd_hbm, s_hbm, out_hbm, s0, s1, v0, v1, ov, sem0, sem1):
        wid = lax.axis_index("s") * 2 + lax.axis_index("c")
        base = wid * NSPW
        pltpu.sync_copy(s_hbm.at[0, pl.ds(base, NSPW)], s0)
        pltpu.sync_copy(s_hbm.at[1, pl.ds(base, NSPW)], s1)
        c0 = pltpu.async_copy(sm_hbm.at[s0], v0, sem0)
        c1 = pltpu.async_copy(sd_hbm.at[s1], v1, sem1)
        c0.wait()
        c1.wait()

        def body(i, c):
            x = v0[pl.ds(i * VL, VL)] + v1[pl.ds(i * VL, VL)]
            ov[pl.ds(i * VL, VL)] = 1.0 / (1.0 + jnp.exp(-x))
            return c

        lax.fori_loop(0, NSPW // VL, body, 0)
        pltpu.sync_copy(ov, out_hbm.at[pl.ds(base, NSPW)])

    return k


def _relayout_idx(idx):
    p, _, l = idx.shape
    x = idx.astype(jnp.int32) + (jnp.arange(l, dtype=jnp.int32) * N2)[None, None, :]
    x = x.transpose(1, 0, 2)
    x = x.reshape(NW, NCHUNK, GR, p * l)
    return x.transpose(0, 1, 3, 2).reshape(NW, NCHUNK * p * l, GR)


def kernel(paths_mm, paths_dd, paths_md, samples, miRNA, disease, Wm, Wd,
           pw1, pw2, fcW, mW0, mb0, mW1, mb1, mW2, mb2):
    idx_a = _relayout_idx(
        jnp.concatenate([paths_mm, paths_dd + Nm], axis=1))
    idx_b = _relayout_idx(paths_md)
    pw1_s = pw1 / float(P)
    pw2_s = pw2 / float(P)
    wt = [(1.0 - ALPHA) * fcW[l].T for l in range(NL)]

    w512 = (mW2 @ mW1 @ mW0).reshape(4, D)
    bias = (mW2 @ (mW1 @ mb0 + mb1) + mb2).reshape(1)
    s_t = samples.astype(jnp.int32).T

    hm = _matmul(miRNA, Wm)
    hd = _matmul(disease, Wd)
    hcat = jnp.concatenate([hm, hd], axis=0)

    gather_a = _make_gather_combine(NCHUNK * P * L1, L1)
    gather_b = _make_gather_combine(NCHUNK * P * L2, L2)

    feats_a = hcat
    for l in range(NL):
        r = gather_a(_prescale(feats_a, pw1_s[l]), idx_a)
        feats_a = _fc(r, wt[l], hcat)

    feats_b = hcat
    for l in range(NL):
        r = gather_b(_prescale(feats_b, pw2_s[l]), idx_b)
        feats_b = _fc(r, wt[l], hcat)

    score2 = _score(feats_a, feats_b, w512, bias)
    out = _make_sample_kernel()(score2[0], score2[1], s_t)
    return out.reshape(NS, 1)


# speedup vs baseline: 12.4158x; 1.1330x over previous
import functools

import jax
import jax.numpy as jnp
from jax import lax
from jax.experimental import pallas as pl
from jax.experimental.pallas import tpu as pltpu
from jax.experimental.pallas import tpu_sc as plsc

Nm, Nd, D = 4096, 4096, 128
P, L1, L2, NL = 8, 4, 8, 2
NS = 16384
ALPHA = 0.1

N2 = Nm + Nd
NW = 32
NPW = N2 // NW
GR = 128
NCHUNK = NPW // GR
VL = 16
NSPW = NS // NW


def _matmul_kernel(a_ref, b_ref, o_ref):
    o_ref[...] = jnp.dot(a_ref[...], b_ref[...],
                         preferred_element_type=jnp.float32)


def _matmul(a, b):
    m, k = a.shape
    _, n = b.shape
    bm = 256
    return pl.pallas_call(
        _matmul_kernel,
        grid=(m // bm,),
        in_specs=[
            pl.BlockSpec((bm, k), lambda i: (i, 0)),
            pl.BlockSpec((k, n), lambda i: (0, 0)),
        ],
        out_specs=pl.BlockSpec((bm, n), lambda i: (i, 0)),
        out_shape=jax.ShapeDtypeStruct((m, n), jnp.float32),
    )(a, b)


def _fc_kernel(r_ref, w_ref, h_ref, o_ref):
    f = jnp.dot(r_ref[...], w_ref[...], preferred_element_type=jnp.float32)
    o_ref[...] = ALPHA * h_ref[...] + jnp.maximum(f, 0.0)


def _fc(r, wt, h):
    bm = 1024
    return pl.pallas_call(
        _fc_kernel,
        grid=(N2 // bm,),
        in_specs=[
            pl.BlockSpec((bm, D), lambda i: (i, 0)),
            pl.BlockSpec((D, D), lambda i: (0, 0)),
            pl.BlockSpec((bm, D), lambda i: (i, 0)),
        ],
        out_specs=pl.BlockSpec((bm, D), lambda i: (i, 0)),
        out_shape=jax.ShapeDtypeStruct((N2, D), jnp.float32),
    )(r, wt, h)


def _prescale_kernel(f_ref, w_ref, o_ref):
    lsel = pl.program_id(0)
    x = f_ref[...] * w_ref[pl.ds(lsel, 1), :]
    bits = lax.bitcast_convert_type(x.astype(jnp.bfloat16), jnp.uint16)
    lo = bits[:, 0:D // 2].astype(jnp.uint32)
    hi = bits[:, D // 2:D].astype(jnp.uint32)
    o_ref[...] = lax.bitcast_convert_type(
        jnp.bitwise_or(lax.shift_left(hi, jnp.uint32(16)), lo), jnp.int32)


def _prescale(feats, pw_s):
    nl = pw_s.shape[0]
    bm = 1024
    nb = N2 // bm
    return pl.pallas_call(
        _prescale_kernel,
        grid=(nl, nb),
        in_specs=[
            pl.BlockSpec((bm, D), lambda l, i: (i, 0)),
            pl.BlockSpec((nl, D), lambda l, i: (0, 0)),
        ],
        out_specs=pl.BlockSpec((bm, D // 2), lambda l, i: (l * nb + i, 0)),
        out_shape=jax.ShapeDtypeStruct((nl * N2, D // 2), jnp.int32),
    )(feats, pw_s)


def _score_kernel(fa_ref, fb_ref, w_ref, b_ref, o_ref):
    dn = (((1,), (1,)), ((), ()))
    pa = lax.dot_general(w_ref[...], fa_ref[...], dn,
                         preferred_element_type=jnp.float32)
    pb = lax.dot_general(w_ref[...], fb_ref[...], dn,
                         preferred_element_type=jnp.float32)
    b = b_ref[0]
    o_ref[0:1, :] = pa[0:1, 0:Nm] + pb[1:2, 0:Nm] + b
    o_ref[1:2, :] = pa[2:3, Nm:N2] + pb[3:4, Nm:N2] + b


def _score(fa, fb, w4, bvec):
    return pl.pallas_call(
        _score_kernel,
        in_specs=[
            pl.BlockSpec((N2, D), lambda: (0, 0)),
            pl.BlockSpec((N2, D), lambda: (0, 0)),
            pl.BlockSpec((4, D), lambda: (0, 0)),
            pl.BlockSpec(memory_space=pltpu.SMEM),
        ],
        out_specs=pl.BlockSpec((2, Nm), lambda: (0, 0)),
        out_shape=jax.ShapeDtypeStruct((2, Nm), jnp.float32),
    )(fa, fb, w4, bvec)


@functools.lru_cache(maxsize=None)
def _make_gather_combine(nstream, nl):
    pl_per_chunk = nstream // NCHUNK
    shift = pl_per_chunk.bit_length() - 1
    mesh = plsc.VectorSubcoreMesh(core_axis_name="c", subcore_axis_name="s")

    @functools.partial(
        pl.kernel,
        out_type=jax.ShapeDtypeStruct((N2, D), jnp.float32),
        mesh=mesh,
        compiler_params=pltpu.CompilerParams(use_tc_tiling_on_sc=False),
        scratch_types=[
            pltpu.VMEM((nstream, GR), jnp.int32),
            pltpu.VMEM((GR, D // 2), jnp.int32),
            pltpu.VMEM((GR, D // 2), jnp.int32),
            pltpu.VMEM((NPW, D), jnp.float32),
            pltpu.SemaphoreType.DMA,
            pltpu.SemaphoreType.DMA,
        ],
    )
    def k(table, idx_hbm, out_hbm, idx_v, buf0, buf1, acc, sem0, sem1):
        wid = lax.axis_index("s") * 2 + lax.axis_index("c")
        base = wid * NPW
        pltpu.sync_copy(idx_hbm.at[wid], idx_v)

        zero = jnp.zeros((VL,), jnp.float32)

        @functools.partial(plsc.parallel_loop, 0, NPW, unroll=4)
        def _(r):
            row = acc.at[r]
            for kk in range(D // VL):
                row[pl.ds(kk * VL, VL)] = zero

        def fire(s, buf, sem):
            pltpu.make_async_copy(table.at[idx_v.at[s]], buf, sem).start()

        def drain(buf, sem):
            pltpu.make_async_copy(table.at[idx_v.at[0]], buf, sem).wait()

        himask = jnp.full((VL,), -65536, jnp.int32)

        def accum(s, buf):
            nb = lax.shift_right_logical(s, shift) * GR

            @functools.partial(plsc.parallel_loop, 0, GR, unroll=4)
            def _(r):
                dst = acc.at[nb + r]
                src = buf.at[r]
                for kk in range(D // (2 * VL)):
                    v = src[pl.ds(kk * VL, VL)]
                    lo = plsc.bitcast(lax.shift_left(v, 16), jnp.float32)
                    hi = plsc.bitcast(v & himask, jnp.float32)
                    plsc.addupdate(dst.at[pl.ds(kk * VL, VL)], lo)
                    plsc.addupdate(dst.at[pl.ds(D // 2 + kk * VL, VL)], hi)

        fire(0, buf0, sem0)

        def pair(i, c):
            s = 2 * i
            fire(s + 1, buf1, sem1)
            drain(buf0, sem0)
            accum(s, buf0)

            @pl.when(s + 2 < nstream)
            def _():
                fire(s + 2, buf0, sem0)

            drain(buf1, sem1)
            accum(s + 1, buf1)
            return c

        lax.fori_loop(0, nstream // 2, pair, 0)
        pltpu.sync_copy(acc, out_hbm.at[pl.ds(base, NPW)])

    return k


@functools.lru_cache(maxsize=None)
def _make_sample_kernel():
    mesh = plsc.VectorSubcoreMesh(core_axis_name="c", subcore_axis_name="s")

    @functools.partial(
        pl.kernel,
        out_type=jax.ShapeDtypeStruct((NS,), jnp.float32),
        mesh=mesh,
        scratch_types=[
            pltpu.VMEM((NSPW,), jnp.int32),
            pltpu.VMEM((NSPW,), jnp.int32),
            pltpu.VMEM((NSPW,), jnp.float32),
            pltpu.VMEM((NSPW,), jnp.float32),
            pltpu.VMEM((NSPW,), jnp.float32),
            pltpu.SemaphoreType.DMA,
            pltpu.SemaphoreType.DMA,
        ],
    )
    def k(sm_hbm, sd_hbm, s_hbm, out_hbm, s0, s1, v0, v1, ov, sem0, sem1):
        wid = lax.axis_index("s") * 2 + lax.axis_index("c")
        base = wid * NSPW
        pltpu.sync_copy(s_hbm.at[0, pl.ds(base, NSPW)], s0)
        pltpu.sync_copy(s_hbm.at[1, pl.ds(base, NSPW)], s1)
        c0 = pltpu.async_copy(sm_hbm.at[s0], v0, sem0)
        c1 = pltpu.async_copy(sd_hbm.at[s1], v1, sem1)
        c0.wait()
        c1.wait()

        def body(i, c):
            x = v0[pl.ds(i * VL, VL)] + v1[pl.ds(i * VL, VL)]
            ov[pl.ds(i * VL, VL)] = 1.0 / (1.0 + jnp.exp(-x))
            return c

        lax.fori_loop(0, NSPW // VL, body, 0)
        pltpu.sync_copy(ov, out_hbm.at[pl.ds(base, NSPW)])

    return k


def _relayout_idx(idx):
    p, _, l = idx.shape
    x = idx.astype(jnp.int32) + (jnp.arange(l, dtype=jnp.int32) * N2)[None, None, :]
    x = x.transpose(1, 0, 2)
    x = x.reshape(NW, NCHUNK, GR, p * l)
    return x.transpose(0, 1, 3, 2).reshape(NW, NCHUNK * p * l, GR)


def kernel(paths_mm, paths_dd, paths_md, samples, miRNA, disease, Wm, Wd,
           pw1, pw2, fcW, mW0, mb0, mW1, mb1, mW2, mb2):
    idx_a = _relayout_idx(
        jnp.concatenate([paths_mm, paths_dd + Nm], axis=1))
    idx_b = _relayout_idx(paths_md)
    pw1_s = pw1 / float(P)
    pw2_s = pw2 / float(P)
    wt = [(1.0 - ALPHA) * fcW[l].T for l in range(NL)]

    w512 = (mW2 @ mW1 @ mW0).reshape(4, D)
    bias = (mW2 @ (mW1 @ mb0 + mb1) + mb2).reshape(1)
    s_t = samples.astype(jnp.int32).T

    hm = _matmul(miRNA, Wm)
    hd = _matmul(disease, Wd)
    hcat = jnp.concatenate([hm, hd], axis=0)

    gather_a = _make_gather_combine(NCHUNK * P * L1, L1)
    gather_b = _make_gather_combine(NCHUNK * P * L2, L2)

    feats_a = hcat
    for l in range(NL):
        r = gather_a(_prescale(feats_a, pw1_s[l]), idx_a)
        feats_a = _fc(r, wt[l], hcat)

    feats_b = hcat
    for l in range(NL):
        r = gather_b(_prescale(feats_b, pw2_s[l]), idx_b)
        feats_b = _fc(r, wt[l], hcat)

    score2 = _score(feats_a, feats_b, w512, bias)
    out = _make_sample_kernel()(score2[0], score2[1], s_t)
    return out.reshape(NS, 1)


# speedup vs baseline: 12.4453x; 1.0024x over previous
import functools

import jax
import jax.numpy as jnp
from jax import lax
from jax.experimental import pallas as pl
from jax.experimental.pallas import tpu as pltpu
from jax.experimental.pallas import tpu_sc as plsc

Nm, Nd, D = 4096, 4096, 128
P, L1, L2, NL = 8, 4, 8, 2
NS = 16384
ALPHA = 0.1

N2 = Nm + Nd
NW = 32
NPW = N2 // NW
GR = 128
NCHUNK = NPW // GR
VL = 16
NSPW = NS // NW


def _matmul_kernel(a_ref, b_ref, o_ref):
    o_ref[...] = jnp.dot(a_ref[...], b_ref[...],
                         preferred_element_type=jnp.float32)


def _matmul(a, b):
    m, k = a.shape
    _, n = b.shape
    bm = 256
    return pl.pallas_call(
        _matmul_kernel,
        grid=(m // bm,),
        in_specs=[
            pl.BlockSpec((bm, k), lambda i: (i, 0)),
            pl.BlockSpec((k, n), lambda i: (0, 0)),
        ],
        out_specs=pl.BlockSpec((bm, n), lambda i: (i, 0)),
        out_shape=jax.ShapeDtypeStruct((m, n), jnp.float32),
    )(a, b)


def _fc_kernel(r_ref, w_ref, h_ref, o_ref):
    f = jnp.dot(r_ref[...], w_ref[...], preferred_element_type=jnp.float32)
    o_ref[...] = ALPHA * h_ref[...] + jnp.maximum(f, 0.0)


def _fc(r, wt, h):
    bm = 1024
    return pl.pallas_call(
        _fc_kernel,
        grid=(N2 // bm,),
        in_specs=[
            pl.BlockSpec((bm, D), lambda i: (i, 0)),
            pl.BlockSpec((D, D), lambda i: (0, 0)),
            pl.BlockSpec((bm, D), lambda i: (i, 0)),
        ],
        out_specs=pl.BlockSpec((bm, D), lambda i: (i, 0)),
        out_shape=jax.ShapeDtypeStruct((N2, D), jnp.float32),
    )(r, wt, h)


def _prescale_kernel(f_ref, w_ref, o_ref):
    lsel = pl.program_id(0)
    x = f_ref[...] * w_ref[pl.ds(lsel, 1), :]
    bits = lax.bitcast_convert_type(x.astype(jnp.bfloat16), jnp.uint16)
    lo = bits[:, 0:D // 2].astype(jnp.uint32)
    hi = bits[:, D // 2:D].astype(jnp.uint32)
    o_ref[...] = lax.bitcast_convert_type(
        jnp.bitwise_or(lax.shift_left(hi, jnp.uint32(16)), lo), jnp.int32)


def _prescale(feats, pw_s):
    nl = pw_s.shape[0]
    bm = 1024
    nb = N2 // bm
    return pl.pallas_call(
        _prescale_kernel,
        grid=(nl, nb),
        in_specs=[
            pl.BlockSpec((bm, D), lambda l, i: (i, 0)),
            pl.BlockSpec((nl, D), lambda l, i: (0, 0)),
        ],
        out_specs=pl.BlockSpec((bm, D // 2), lambda l, i: (l * nb + i, 0)),
        out_shape=jax.ShapeDtypeStruct((nl * N2, D // 2), jnp.int32),
    )(feats, pw_s)


def _score_kernel(fa_ref, fb_ref, w_ref, b_ref, o_ref):
    dn = (((1,), (1,)), ((), ()))
    pa = lax.dot_general(w_ref[...], fa_ref[...], dn,
                         preferred_element_type=jnp.float32)
    pb = lax.dot_general(w_ref[...], fb_ref[...], dn,
                         preferred_element_type=jnp.float32)
    b = b_ref[0]
    o_ref[0:1, :] = pa[0:1, 0:Nm] + pb[1:2, 0:Nm] + b
    o_ref[1:2, :] = pa[2:3, Nm:N2] + pb[3:4, Nm:N2] + b


def _score(fa, fb, w4, bvec):
    return pl.pallas_call(
        _score_kernel,
        in_specs=[
            pl.BlockSpec((N2, D), lambda: (0, 0)),
            pl.BlockSpec((N2, D), lambda: (0, 0)),
            pl.BlockSpec((4, D), lambda: (0, 0)),
            pl.BlockSpec(memory_space=pltpu.SMEM),
        ],
        out_specs=pl.BlockSpec((2, Nm), lambda: (0, 0)),
        out_shape=jax.ShapeDtypeStruct((2, Nm), jnp.float32),
    )(fa, fb, w4, bvec)


@functools.lru_cache(maxsize=None)
def _make_gather_combine(nstream, nl):
    pl_per_chunk = nstream // NCHUNK
    shift = pl_per_chunk.bit_length() - 1
    mesh = plsc.VectorSubcoreMesh(core_axis_name="c", subcore_axis_name="s")

    @functools.partial(
        pl.kernel,
        out_type=jax.ShapeDtypeStruct((N2, D), jnp.float32),
        mesh=mesh,
        compiler_params=pltpu.CompilerParams(use_tc_tiling_on_sc=False),
        scratch_types=[
            pltpu.VMEM((nstream, GR), jnp.int32),
            pltpu.VMEM((GR, D // 2), jnp.int32),
            pltpu.VMEM((GR, D // 2), jnp.int32),
            pltpu.VMEM((NPW, D), jnp.float32),
            pltpu.SemaphoreType.DMA,
            pltpu.SemaphoreType.DMA,
        ],
    )
    def k(table, idx_hbm, out_hbm, idx_v, buf0, buf1, acc, sem0, sem1):
        wid = lax.axis_index("s") * 2 + lax.axis_index("c")
        base = wid * NPW
        pltpu.sync_copy(idx_hbm.at[wid], idx_v)

        zero = jnp.zeros((VL,), jnp.float32)

        @functools.partial(plsc.parallel_loop, 0, NPW, unroll=4)
        def _(r):
            row = acc.at[r]
            for kk in range(D // VL):
                row[pl.ds(kk * VL, VL)] = zero

        def fire(s, buf, sem):
            pltpu.make_async_copy(table.at[idx_v.at[s]], buf, sem).start()

        def drain(buf, sem):
            pltpu.make_async_copy(table.at[idx_v.at[0]], buf, sem).wait()

        himask = jnp.full((VL,), -65536, jnp.int32)

        def accum(s, buf):
            nb = lax.shift_right_logical(s, shift) * GR

            @functools.partial(plsc.parallel_loop, 0, GR, unroll=4)
            def _(r):
                dst = acc.at[nb + r]
                src = buf.at[r]
                for kk in range(D // (2 * VL)):
                    v = src[pl.ds(kk * VL, VL)]
                    lo = plsc.bitcast(lax.shift_left(v, 16), jnp.float32)
                    hi = plsc.bitcast(v & himask, jnp.float32)
                    plsc.addupdate(dst.at[pl.ds(kk * VL, VL)], lo)
                    plsc.addupdate(dst.at[pl.ds(D // 2 + kk * VL, VL)], hi)

        fire(0, buf0, sem0)

        def pair(i, c):
            s = 2 * i
            fire(s + 1, buf1, sem1)
            drain(buf0, sem0)
            accum(s, buf0)

            @pl.when(s + 2 < nstream)
            def _():
                fire(s + 2, buf0, sem0)

            drain(buf1, sem1)
            accum(s + 1, buf1)
            return c

        lax.fori_loop(0, nstream // 2, pair, 0)
        pltpu.sync_copy(acc, out_hbm.at[pl.ds(base, NPW)])

    return k


@functools.lru_cache(maxsize=None)
def _make_sample_kernel():
    mesh = plsc.VectorSubcoreMesh(core_axis_name="c", subcore_axis_name="s")

    @functools.partial(
        pl.kernel,
        out_type=jax.ShapeDtypeStruct((NS,), jnp.float32),
        mesh=mesh,
        scratch_types=[
            pltpu.VMEM((NSPW,), jnp.int32),
            pltpu.VMEM((NSPW,), jnp.int32),
            pltpu.VMEM((NSPW,), jnp.float32),
            pltpu.VMEM((NSPW,), jnp.float32),
            pltpu.VMEM((NSPW,), jnp.float32),
            pltpu.SemaphoreType.DMA,
            pltpu.SemaphoreType.DMA,
        ],
    )
    def k(sm_hbm, sd_hbm, s_hbm, out_hbm, s0, s1, v0, v1, ov, sem0, sem1):
        wid = lax.axis_index("s") * 2 + lax.axis_index("c")
        base = wid * NSPW
        pltpu.sync_copy(s_hbm.at[0, pl.ds(base, NSPW)], s0)
        pltpu.sync_copy(s_hbm.at[1, pl.ds(base, NSPW)], s1)
        c0 = pltpu.async_copy(sm_hbm.at[s0], v0, sem0)
        c1 = pltpu.async_copy(sd_hbm.at[s1], v1, sem1)
        c0.wait()
        c1.wait()

        def body(i, c):
            x = v0[pl.ds(i * VL, VL)] + v1[pl.ds(i * VL, VL)]
            ov[pl.ds(i * VL, VL)] = 1.0 / (1.0 + jnp.exp(-x))
            return c

        lax.fori_loop(0, NSPW // VL, body, 0)
        pltpu.sync_copy(ov, out_hbm.at[pl.ds(base, NSPW)])

    return k


def _relayout_idx(idx):
    p, _, l = idx.shape
    x = idx.astype(jnp.int32) + (jnp.arange(l, dtype=jnp.int32) * N2)[None, None, :]
    x = x.transpose(1, 0, 2)
    x = x.reshape(NW, NCHUNK, GR, p * l)
    return x.transpose(0, 1, 3, 2).reshape(NW, NCHUNK * p * l, GR)


def kernel(paths_mm, paths_dd, paths_md, samples, miRNA, disease, Wm, Wd,
           pw1, pw2, fcW, mW0, mb0, mW1, mb1, mW2, mb2):
    idx_a = _relayout_idx(
        jnp.concatenate([paths_mm, paths_dd + Nm], axis=1))
    idx_b = _relayout_idx(paths_md)
    pw1_s = pw1 / float(P)
    pw2_s = pw2 / float(P)
    wt = [(1.0 - ALPHA) * fcW[l].T for l in range(NL)]

    w512 = (mW2 @ mW1 @ mW0).reshape(4, D)
    bias = (mW2 @ (mW1 @ mb0 + mb1) + mb2).reshape(1)
    s_t = samples.astype(jnp.int32).T

    hm = _matmul(miRNA, Wm)
    hd = _matmul(disease, Wd)
    hcat = jnp.concatenate([hm, hd], axis=0)

    gather_a = _make_gather_combine(NCHUNK * P * L1, L1)
    gather_b = _make_gather_combine(NCHUNK * P * L2, L2)

    feats_a = hcat
    feats_b = hcat
    for l in range(NL):
        ts_a = _prescale(feats_a, pw1_s[l])
        ts_b = _prescale(feats_b, pw2_s[l])
        r_a = gather_a(ts_a, idx_a)
        r_b = gather_b(ts_b, idx_b)
        feats_a = _fc(r_a, wt[l], hcat)
        feats_b = _fc(r_b, wt[l], hcat)

    score2 = _score(feats_a, feats_b, w512, bias)
    out = _make_sample_kernel()(score2[0], score2[1], s_t)
    return out.reshape(NS, 1)
